# Initial kernel scaffold; baseline (speedup 1.0000x reference)
import jax
import jax.numpy as jnp
from jax.experimental import pallas as pl


def kernel(feat_data, edge_index, W1, b1, W2, b2):
    raise NotImplementedError("write your pallas kernel here")


# speedup vs baseline: 7.8967x; 7.8967x over previous
import functools

import jax
import jax.numpy as jnp
from jax import lax
from jax.experimental import pallas as pl
from jax.experimental.pallas import tpu as pltpu
from jax.experimental.pallas import tpu_sc as plsc

N_NODES = 10000
D = 128
DH = D // 2

NC = 2
NS = 16
NW = NC * NS
C = 128

NN = 10240
RPT = NN // NS


def _pad_edges(src, dst, n_edges):
    epw = ((n_edges + NW - 1) // NW + 8 * C - 1) // (8 * C) * (8 * C)
    ep = epw * NW
    pad = ep - n_edges
    pad_ids = N_NODES + (jnp.arange(pad, dtype=jnp.int32) % (NN - N_NODES))
    src_p = jnp.concatenate([src, pad_ids])
    dst_p = jnp.concatenate([dst, pad_ids])
    return src_p.reshape(ep // C, C), dst_p.reshape(ep // C, C), ep // C


_MESH = plsc.VectorSubcoreMesh(core_axis_name="c", subcore_axis_name="s")


def _sc_degrees(src2d, dst2d, zeros1, nch_tot):
    nch = nch_tot // NW

    @functools.partial(
        pl.kernel,
        out_type=jax.ShapeDtypeStruct((NC, 2, NN), jnp.float32),
        mesh=_MESH,
        scratch_types=[
            pltpu.VMEM((nch, C), jnp.int32),
            pltpu.VMEM((nch, C), jnp.int32),
            pltpu.VMEM((C,), jnp.float32),
            pltpu.VMEM_SHARED((NN,), jnp.float32),
            pltpu.VMEM_SHARED((NN,), jnp.float32),
        ],
    )
    def deg_kernel(src_hbm, dst_hbm, z_hbm, out_hbm,
                   src_v, dst_v, ones_v, dego_sp, degi_sp):
        c = lax.axis_index("c")
        s = lax.axis_index("s")
        w = c * NS + s
        pltpu.sync_copy(z_hbm.at[pl.ds(0, RPT)], dego_sp.at[pl.ds(s * RPT, RPT)])
        pltpu.sync_copy(z_hbm.at[pl.ds(0, RPT)], degi_sp.at[pl.ds(s * RPT, RPT)])
        pltpu.sync_copy(src_hbm.at[pl.ds(w * nch, nch)], src_v)
        pltpu.sync_copy(dst_hbm.at[pl.ds(w * nch, nch)], dst_v)
        for i in range(C // 16):
            ones_v[pl.ds(16 * i, 16)] = jnp.full((16,), 1.0, jnp.float32)
        plsc.subcore_barrier()

        def body(g, carry):
            pltpu.sync_copy(ones_v, dego_sp.at[src_v.at[g]], add=True)
            pltpu.sync_copy(ones_v, degi_sp.at[dst_v.at[g]], add=True)
            return carry

        lax.fori_loop(0, nch, body, 0)
        plsc.subcore_barrier()
        pltpu.sync_copy(dego_sp.at[pl.ds(s * RPT, RPT)],
                        out_hbm.at[c, 0, pl.ds(s * RPT, RPT)])
        pltpu.sync_copy(degi_sp.at[pl.ds(s * RPT, RPT)],
                        out_hbm.at[c, 1, pl.ds(s * RPT, RPT)])

    return deg_kernel(src2d, dst2d, zeros1)


def _sc_aggregate(h_lo, h_hi, src2d, dst2d, zeros2, nch_tot):
    nch = nch_tot // NS

    @functools.partial(
        pl.kernel,
        out_type=jax.ShapeDtypeStruct((NC, NN, DH), jnp.float32),
        mesh=_MESH,
        compiler_params=pltpu.CompilerParams(use_tc_tiling_on_sc=False),
        scratch_types=[
            pltpu.VMEM((nch, C), jnp.int32),
            pltpu.VMEM((nch, C), jnp.int32),
            pltpu.VMEM((C, DH), jnp.float32),
            pltpu.VMEM((C, DH), jnp.float32),
            pltpu.VMEM_SHARED((NN, DH), jnp.float32),
            pltpu.SemaphoreType.DMA,
            pltpu.SemaphoreType.DMA,
        ],
    )
    def agg_kernel(hlo_hbm, hhi_hbm, src_hbm, dst_hbm, z_hbm, out_hbm,
                   src_v, dst_v, rows_a, rows_b, agg_sp, sem_a, sem_b):
        c = lax.axis_index("c")
        s = lax.axis_index("s")
        pltpu.sync_copy(z_hbm, agg_sp.at[pl.ds(s * RPT, RPT)])
        pltpu.sync_copy(src_hbm.at[pl.ds(s * nch, nch)], src_v)
        pltpu.sync_copy(dst_hbm.at[pl.ds(s * nch, nch)], dst_v)
        plsc.subcore_barrier()

        def one_core(h_hbm):
            def body(k, carry):
                g0 = 2 * k
                ca = pltpu.async_copy(h_hbm.at[src_v.at[g0]], rows_a, sem_a)
                cb = pltpu.async_copy(h_hbm.at[src_v.at[g0 + 1]], rows_b, sem_b)
                ca.wait()
                pltpu.sync_copy(rows_a, agg_sp.at[dst_v.at[g0]], add=True)
                cb.wait()
                pltpu.sync_copy(rows_b, agg_sp.at[dst_v.at[g0 + 1]], add=True)
                return carry
            lax.fori_loop(0, nch // 2, body, 0)

        @pl.when(c == 0)
        def _():
            one_core(hlo_hbm)

        @pl.when(c == 1)
        def _():
            one_core(hhi_hbm)

        plsc.subcore_barrier()
        pltpu.sync_copy(agg_sp.at[pl.ds(s * RPT, RPT)],
                        out_hbm.at[c, pl.ds(s * RPT, RPT)])

    return agg_kernel(h_lo, h_hi, src2d, dst2d, zeros2)


_BM = 1024


def _tc_prep(feat_pad, dego2, degi2):
    def body(f_ref, do_ref, di_ref, h1_ref, ns_ref, nd_ref):
        do = do_ref[0] + do_ref[1]
        di = di_ref[0] + di_ref[1]
        ns = lax.rsqrt(jnp.maximum(do, 1.0))
        nd = lax.rsqrt(jnp.maximum(di, 1.0))
        h1_ref[...] = f_ref[...] * ns
        ns_ref[...] = ns
        nd_ref[...] = nd

    return pl.pallas_call(
        body,
        grid=(NN // _BM,),
        in_specs=[
            pl.BlockSpec((_BM, D), lambda i: (i, 0)),
            pl.BlockSpec((NC, _BM, 1), lambda i: (0, i, 0)),
            pl.BlockSpec((NC, _BM, 1), lambda i: (0, i, 0)),
        ],
        out_specs=[
            pl.BlockSpec((_BM, D), lambda i: (i, 0)),
            pl.BlockSpec((_BM, 1), lambda i: (i, 0)),
            pl.BlockSpec((_BM, 1), lambda i: (i, 0)),
        ],
        out_shape=[
            jax.ShapeDtypeStruct((NN, D), jnp.float32),
            jax.ShapeDtypeStruct((NN, 1), jnp.float32),
            jax.ShapeDtypeStruct((NN, 1), jnp.float32),
        ],
    )(feat_pad, dego2, degi2)


def _tc_dense(aggp, nd_col, W, b_row, ns_col=None):
    def body(a_ref, nd_ref, w_ref, b_ref, *rest):
        if ns_col is not None:
            ns_ref, o_ref = rest
        else:
            (o_ref,) = rest
        a = jnp.concatenate([a_ref[0], a_ref[1]], axis=1) * nd_ref[...]
        y = jnp.dot(a, w_ref[...], preferred_element_type=jnp.float32)
        y = y + b_ref[...]
        if ns_col is not None:
            y = jnp.maximum(y, 0.0) * ns_ref[...]
        o_ref[...] = y

    in_specs = [
        pl.BlockSpec((NC, _BM, DH), lambda i: (0, i, 0)),
        pl.BlockSpec((_BM, 1), lambda i: (i, 0)),
        pl.BlockSpec((D, D), lambda i: (0, 0)),
        pl.BlockSpec((1, D), lambda i: (0, 0)),
    ]
    args = [aggp, nd_col, W, b_row]
    if ns_col is not None:
        in_specs.append(pl.BlockSpec((_BM, 1), lambda i: (i, 0)))
        args.append(ns_col)
    return pl.pallas_call(
        body,
        grid=(NN // _BM,),
        in_specs=in_specs,
        out_specs=pl.BlockSpec((_BM, D), lambda i: (i, 0)),
        out_shape=jax.ShapeDtypeStruct((NN, D), jnp.float32),
    )(*args)


def kernel(feat_data, edge_index, W1, b1, W2, b2):
    n_edges = edge_index.shape[1]
    src = edge_index[0].astype(jnp.int32)
    dst = edge_index[1].astype(jnp.int32)
    src2d, dst2d, nch_tot = _pad_edges(src, dst, n_edges)

    feat_pad = jnp.pad(feat_data, ((0, NN - N_NODES), (0, 0)))
    zeros1 = jnp.zeros((NN,), jnp.float32)
    zeros2 = jnp.zeros((RPT, DH), jnp.float32)
    b1_row = b1.reshape(1, D)
    b2_row = b2.reshape(1, D)

    degp = _sc_degrees(src2d, dst2d, zeros1, nch_tot)
    dego2 = degp[:, 0, :].reshape(NC, NN, 1)
    degi2 = degp[:, 1, :].reshape(NC, NN, 1)

    h1, ns_col, nd_col = _tc_prep(feat_pad, dego2, degi2)
    aggp1 = _sc_aggregate(h1[:, :DH], h1[:, DH:], src2d, dst2d, zeros2,
                          nch_tot)
    h2 = _tc_dense(aggp1, nd_col, W1, b1_row, ns_col=ns_col)
    aggp2 = _sc_aggregate(h2[:, :DH], h2[:, DH:], src2d, dst2d, zeros2,
                          nch_tot)
    out = _tc_dense(aggp2, nd_col, W2, b2_row)
    return out[:N_NODES]


# speedup vs baseline: 10.2523x; 1.2983x over previous
import functools

import jax
import jax.numpy as jnp
from jax import lax
from jax.experimental import pallas as pl
from jax.experimental.pallas import tpu as pltpu
from jax.experimental.pallas import tpu_sc as plsc

N_NODES = 10000
D = 128
DH = D // 2

NC = 2
NS = 16
NW = NC * NS
C = 128

NN = 10240
RPT = NN // NS


def _pad_edges(src, dst, n_edges):
    epw = ((n_edges + NW - 1) // NW + 8 * C - 1) // (8 * C) * (8 * C)
    ep = epw * NW
    pad = ep - n_edges
    pad_ids = N_NODES + (jnp.arange(pad, dtype=jnp.int32) % (NN - N_NODES))
    src_p = jnp.concatenate([src, pad_ids])
    dst_p = jnp.concatenate([dst, pad_ids])
    return src_p.reshape(ep // C, C), dst_p.reshape(ep // C, C), ep // C


_MESH = plsc.VectorSubcoreMesh(core_axis_name="c", subcore_axis_name="s")


def _sc_degrees(src2d, dst2d, zeros1, nch_tot):
    nch = nch_tot // NW

    @functools.partial(
        pl.kernel,
        out_type=jax.ShapeDtypeStruct((NC, 2, NN), jnp.float32),
        mesh=_MESH,
        scratch_types=[
            pltpu.VMEM((nch, C), jnp.int32),
            pltpu.VMEM((nch, C), jnp.int32),
            pltpu.VMEM((C,), jnp.float32),
            pltpu.VMEM_SHARED((NN,), jnp.float32),
            pltpu.VMEM_SHARED((NN,), jnp.float32),
        ],
    )
    def deg_kernel(src_hbm, dst_hbm, z_hbm, out_hbm,
                   src_v, dst_v, ones_v, dego_sp, degi_sp):
        c = lax.axis_index("c")
        s = lax.axis_index("s")
        w = c * NS + s
        pltpu.sync_copy(z_hbm.at[pl.ds(0, RPT)], dego_sp.at[pl.ds(s * RPT, RPT)])
        pltpu.sync_copy(z_hbm.at[pl.ds(0, RPT)], degi_sp.at[pl.ds(s * RPT, RPT)])
        pltpu.sync_copy(src_hbm.at[pl.ds(w * nch, nch)], src_v)
        pltpu.sync_copy(dst_hbm.at[pl.ds(w * nch, nch)], dst_v)
        for i in range(C // 16):
            ones_v[pl.ds(16 * i, 16)] = jnp.full((16,), 1.0, jnp.float32)
        plsc.subcore_barrier()

        def body(g, carry):
            pltpu.sync_copy(ones_v, dego_sp.at[src_v.at[g]], add=True)
            pltpu.sync_copy(ones_v, degi_sp.at[dst_v.at[g]], add=True)
            return carry

        lax.fori_loop(0, nch, body, 0)
        plsc.subcore_barrier()
        pltpu.sync_copy(dego_sp.at[pl.ds(s * RPT, RPT)],
                        out_hbm.at[c, 0, pl.ds(s * RPT, RPT)])
        pltpu.sync_copy(degi_sp.at[pl.ds(s * RPT, RPT)],
                        out_hbm.at[c, 1, pl.ds(s * RPT, RPT)])

    return deg_kernel(src2d, dst2d, zeros1)


_NBUF = 4


def _sc_aggregate(h_lo, h_hi, src2d, dst2d, zeros2, nch_tot):
    nch = nch_tot // NS

    @functools.partial(
        pl.kernel,
        out_type=jax.ShapeDtypeStruct((NC, NN, DH), jnp.float32),
        mesh=_MESH,
        compiler_params=pltpu.CompilerParams(use_tc_tiling_on_sc=False),
        scratch_types=[
            pltpu.VMEM((nch, C), jnp.int32),
            pltpu.VMEM((nch, C), jnp.int32),
            pltpu.VMEM((_NBUF, C, DH), jnp.float32),
            pltpu.VMEM_SHARED((NN, DH), jnp.float32),
            [pltpu.SemaphoreType.DMA] * _NBUF,
            [pltpu.SemaphoreType.DMA] * _NBUF,
        ],
    )
    def agg_kernel(hlo_hbm, hhi_hbm, src_hbm, dst_hbm, z_hbm, out_hbm,
                   src_v, dst_v, rows, agg_sp, sg, ss):
        c = lax.axis_index("c")
        s = lax.axis_index("s")
        pltpu.sync_copy(z_hbm, agg_sp.at[pl.ds(s * RPT, RPT)])
        pltpu.sync_copy(src_hbm.at[pl.ds(s * nch, nch)], src_v)
        pltpu.sync_copy(dst_hbm.at[pl.ds(s * nch, nch)], dst_v)
        plsc.subcore_barrier()

        def one_core(h_hbm):
            def gather(b, g):
                pltpu.async_copy(h_hbm.at[src_v.at[g]], rows.at[b], sg[b])

            def gather_wait(b, g):
                pltpu.make_async_copy(h_hbm.at[src_v.at[g]], rows.at[b],
                                      sg[b]).wait()

            def scatter(b, g):
                pltpu.async_copy(rows.at[b], agg_sp.at[dst_v.at[g]],
                                 ss[b], add=True)

            def scatter_wait(b, g):
                pltpu.make_async_copy(rows.at[b], agg_sp.at[dst_v.at[g]],
                                      ss[b]).wait()

            for b in range(_NBUF):
                gather(b, b)

            def body(k, carry):
                g0 = _NBUF * k
                for b in range(_NBUF):
                    gather_wait(b, g0 + b)
                    scatter(b, g0 + b)
                for b in range(_NBUF):
                    scatter_wait(b, g0 + b)
                    gather(b, g0 + _NBUF + b)
                return carry

            lax.fori_loop(0, nch // _NBUF - 1, body, 0)
            g0 = nch - _NBUF
            for b in range(_NBUF):
                gather_wait(b, g0 + b)
                scatter(b, g0 + b)
            for b in range(_NBUF):
                scatter_wait(b, g0 + b)

        @pl.when(c == 0)
        def _():
            one_core(hlo_hbm)

        @pl.when(c == 1)
        def _():
            one_core(hhi_hbm)

        plsc.subcore_barrier()
        pltpu.sync_copy(agg_sp.at[pl.ds(s * RPT, RPT)],
                        out_hbm.at[c, pl.ds(s * RPT, RPT)])

    return agg_kernel(h_lo, h_hi, src2d, dst2d, zeros2)


_BM = 1024


def _tc_prep(feat_pad, dego2, degi2):
    def body(f_ref, do_ref, di_ref, h1_ref, ns_ref, nd_ref):
        do = do_ref[0] + do_ref[1]
        di = di_ref[0] + di_ref[1]
        ns = lax.rsqrt(jnp.maximum(do, 1.0))
        nd = lax.rsqrt(jnp.maximum(di, 1.0))
        h1_ref[...] = f_ref[...] * ns
        ns_ref[...] = ns
        nd_ref[...] = nd

    return pl.pallas_call(
        body,
        grid=(NN // _BM,),
        in_specs=[
            pl.BlockSpec((_BM, D), lambda i: (i, 0)),
            pl.BlockSpec((NC, _BM, 1), lambda i: (0, i, 0)),
            pl.BlockSpec((NC, _BM, 1), lambda i: (0, i, 0)),
        ],
        out_specs=[
            pl.BlockSpec((_BM, D), lambda i: (i, 0)),
            pl.BlockSpec((_BM, 1), lambda i: (i, 0)),
            pl.BlockSpec((_BM, 1), lambda i: (i, 0)),
        ],
        out_shape=[
            jax.ShapeDtypeStruct((NN, D), jnp.float32),
            jax.ShapeDtypeStruct((NN, 1), jnp.float32),
            jax.ShapeDtypeStruct((NN, 1), jnp.float32),
        ],
    )(feat_pad, dego2, degi2)


def _tc_dense(aggp, nd_col, W, b_row, ns_col=None):
    def body(a_ref, nd_ref, w_ref, b_ref, *rest):
        if ns_col is not None:
            ns_ref, o_ref = rest
        else:
            (o_ref,) = rest
        a = jnp.concatenate([a_ref[0], a_ref[1]], axis=1) * nd_ref[...]
        y = jnp.dot(a, w_ref[...], preferred_element_type=jnp.float32)
        y = y + b_ref[...]
        if ns_col is not None:
            y = jnp.maximum(y, 0.0) * ns_ref[...]
        o_ref[...] = y

    in_specs = [
        pl.BlockSpec((NC, _BM, DH), lambda i: (0, i, 0)),
        pl.BlockSpec((_BM, 1), lambda i: (i, 0)),
        pl.BlockSpec((D, D), lambda i: (0, 0)),
        pl.BlockSpec((1, D), lambda i: (0, 0)),
    ]
    args = [aggp, nd_col, W, b_row]
    if ns_col is not None:
        in_specs.append(pl.BlockSpec((_BM, 1), lambda i: (i, 0)))
        args.append(ns_col)
    return pl.pallas_call(
        body,
        grid=(NN // _BM,),
        in_specs=in_specs,
        out_specs=pl.BlockSpec((_BM, D), lambda i: (i, 0)),
        out_shape=jax.ShapeDtypeStruct((NN, D), jnp.float32),
    )(*args)


def kernel(feat_data, edge_index, W1, b1, W2, b2):
    n_edges = edge_index.shape[1]
    src = edge_index[0].astype(jnp.int32)
    dst = edge_index[1].astype(jnp.int32)
    src2d, dst2d, nch_tot = _pad_edges(src, dst, n_edges)

    feat_pad = jnp.pad(feat_data, ((0, NN - N_NODES), (0, 0)))
    zeros1 = jnp.zeros((NN,), jnp.float32)
    zeros2 = jnp.zeros((RPT, DH), jnp.float32)
    b1_row = b1.reshape(1, D)
    b2_row = b2.reshape(1, D)

    degp = _sc_degrees(src2d, dst2d, zeros1, nch_tot)
    dego2 = degp[:, 0, :].reshape(NC, NN, 1)
    degi2 = degp[:, 1, :].reshape(NC, NN, 1)

    h1, ns_col, nd_col = _tc_prep(feat_pad, dego2, degi2)
    aggp1 = _sc_aggregate(h1[:, :DH], h1[:, DH:], src2d, dst2d, zeros2,
                          nch_tot)
    h2 = _tc_dense(aggp1, nd_col, W1, b1_row, ns_col=ns_col)
    aggp2 = _sc_aggregate(h2[:, :DH], h2[:, DH:], src2d, dst2d, zeros2,
                          nch_tot)
    out = _tc_dense(aggp2, nd_col, W2, b2_row)
    return out[:N_NODES]


# speedup vs baseline: 10.3886x; 1.0133x over previous
import functools

import jax
import jax.numpy as jnp
from jax import lax
from jax.experimental import pallas as pl
from jax.experimental.pallas import tpu as pltpu
from jax.experimental.pallas import tpu_sc as plsc

N_NODES = 10000
D = 128
DH = D // 2

NC = 2
NS = 16
NW = NC * NS
C = 128

NN = 10240
RPT = NN // NS


def _pad_edges(src, dst, n_edges):
    epw = ((n_edges + NW - 1) // NW + 8 * C - 1) // (8 * C) * (8 * C)
    ep = epw * NW
    pad = ep - n_edges
    pad_ids = N_NODES + (jnp.arange(pad, dtype=jnp.int32) % (NN - N_NODES))
    src_p = jnp.concatenate([src, pad_ids])
    dst_p = jnp.concatenate([dst, pad_ids])
    return src_p.reshape(ep // C, C), dst_p.reshape(ep // C, C), ep // C


_MESH = plsc.VectorSubcoreMesh(core_axis_name="c", subcore_axis_name="s")


def _sc_degrees(src2d, dst2d, zeros1, nch_tot):
    nch = nch_tot // NW

    @functools.partial(
        pl.kernel,
        out_type=jax.ShapeDtypeStruct((NC, 2, NN), jnp.float32),
        mesh=_MESH,
        scratch_types=[
            pltpu.VMEM((nch, C), jnp.int32),
            pltpu.VMEM((nch, C), jnp.int32),
            pltpu.VMEM((C,), jnp.float32),
            pltpu.VMEM_SHARED((NN,), jnp.float32),
            pltpu.VMEM_SHARED((NN,), jnp.float32),
        ],
    )
    def deg_kernel(src_hbm, dst_hbm, z_hbm, out_hbm,
                   src_v, dst_v, ones_v, dego_sp, degi_sp):
        c = lax.axis_index("c")
        s = lax.axis_index("s")
        w = c * NS + s
        pltpu.sync_copy(z_hbm.at[pl.ds(0, RPT)], dego_sp.at[pl.ds(s * RPT, RPT)])
        pltpu.sync_copy(z_hbm.at[pl.ds(0, RPT)], degi_sp.at[pl.ds(s * RPT, RPT)])
        pltpu.sync_copy(src_hbm.at[pl.ds(w * nch, nch)], src_v)
        pltpu.sync_copy(dst_hbm.at[pl.ds(w * nch, nch)], dst_v)
        for i in range(C // 16):
            ones_v[pl.ds(16 * i, 16)] = jnp.full((16,), 1.0, jnp.float32)
        plsc.subcore_barrier()

        def body(g, carry):
            pltpu.sync_copy(ones_v, dego_sp.at[src_v.at[g]], add=True)
            pltpu.sync_copy(ones_v, degi_sp.at[dst_v.at[g]], add=True)
            return carry

        lax.fori_loop(0, nch, body, 0)
        plsc.subcore_barrier()
        pltpu.sync_copy(dego_sp.at[pl.ds(s * RPT, RPT)],
                        out_hbm.at[c, 0, pl.ds(s * RPT, RPT)])
        pltpu.sync_copy(degi_sp.at[pl.ds(s * RPT, RPT)],
                        out_hbm.at[c, 1, pl.ds(s * RPT, RPT)])

    return deg_kernel(src2d, dst2d, zeros1)


_NBUF = 5


def _sc_aggregate(h_lo, h_hi, src2d, dst2d, zeros2, nch_tot):
    nch = nch_tot // NS

    @functools.partial(
        pl.kernel,
        out_type=jax.ShapeDtypeStruct((NC, NN, DH), jnp.float32),
        mesh=_MESH,
        compiler_params=pltpu.CompilerParams(use_tc_tiling_on_sc=False),
        scratch_types=[
            pltpu.VMEM((nch, C), jnp.int32),
            pltpu.VMEM((nch, C), jnp.int32),
            pltpu.VMEM((_NBUF, C, DH), jnp.float32),
            pltpu.VMEM_SHARED((NN, DH), jnp.float32),
            [pltpu.SemaphoreType.DMA] * _NBUF,
            [pltpu.SemaphoreType.DMA] * _NBUF,
        ],
    )
    def agg_kernel(hlo_hbm, hhi_hbm, src_hbm, dst_hbm, z_hbm, out_hbm,
                   src_v, dst_v, rows, agg_sp, sg, ss):
        c = lax.axis_index("c")
        s = lax.axis_index("s")
        pltpu.sync_copy(z_hbm, agg_sp.at[pl.ds(s * RPT, RPT)])
        pltpu.sync_copy(src_hbm.at[pl.ds(s * nch, nch)], src_v)
        pltpu.sync_copy(dst_hbm.at[pl.ds(s * nch, nch)], dst_v)
        plsc.subcore_barrier()

        def one_core(h_hbm):
            def gather(b, g):
                pltpu.async_copy(h_hbm.at[src_v.at[g]], rows.at[b], sg[b])

            def gather_wait(b, g):
                pltpu.make_async_copy(h_hbm.at[src_v.at[g]], rows.at[b],
                                      sg[b]).wait()

            def scatter(b, g):
                pltpu.async_copy(rows.at[b], agg_sp.at[dst_v.at[g]],
                                 ss[b], add=True)

            def scatter_wait(b, g):
                pltpu.make_async_copy(rows.at[b], agg_sp.at[dst_v.at[g]],
                                      ss[b]).wait()

            for b in range(_NBUF):
                gather(b, b)

            def body(k, carry):
                g0 = _NBUF * k
                for b in range(_NBUF):
                    gather_wait(b, g0 + b)
                    scatter(b, g0 + b)
                for b in range(_NBUF):
                    scatter_wait(b, g0 + b)
                    gather(b, g0 + _NBUF + b)
                return carry

            lax.fori_loop(0, nch // _NBUF - 1, body, 0)
            g0 = nch - _NBUF
            for b in range(_NBUF):
                gather_wait(b, g0 + b)
                scatter(b, g0 + b)
            for b in range(_NBUF):
                scatter_wait(b, g0 + b)

        @pl.when(c == 0)
        def _():
            one_core(hlo_hbm)

        @pl.when(c == 1)
        def _():
            one_core(hhi_hbm)

        plsc.subcore_barrier()
        pltpu.sync_copy(agg_sp.at[pl.ds(s * RPT, RPT)],
                        out_hbm.at[c, pl.ds(s * RPT, RPT)])

    return agg_kernel(h_lo, h_hi, src2d, dst2d, zeros2)


_BM = 1024


def _tc_prep(feat_pad, dego2, degi2):
    def body(f_ref, do_ref, di_ref, h1_ref, ns_ref, nd_ref):
        do = do_ref[0] + do_ref[1]
        di = di_ref[0] + di_ref[1]
        ns = lax.rsqrt(jnp.maximum(do, 1.0))
        nd = lax.rsqrt(jnp.maximum(di, 1.0))
        h1_ref[...] = f_ref[...] * ns
        ns_ref[...] = ns
        nd_ref[...] = nd

    return pl.pallas_call(
        body,
        grid=(NN // _BM,),
        in_specs=[
            pl.BlockSpec((_BM, D), lambda i: (i, 0)),
            pl.BlockSpec((NC, _BM, 1), lambda i: (0, i, 0)),
            pl.BlockSpec((NC, _BM, 1), lambda i: (0, i, 0)),
        ],
        out_specs=[
            pl.BlockSpec((_BM, D), lambda i: (i, 0)),
            pl.BlockSpec((_BM, 1), lambda i: (i, 0)),
            pl.BlockSpec((_BM, 1), lambda i: (i, 0)),
        ],
        out_shape=[
            jax.ShapeDtypeStruct((NN, D), jnp.float32),
            jax.ShapeDtypeStruct((NN, 1), jnp.float32),
            jax.ShapeDtypeStruct((NN, 1), jnp.float32),
        ],
    )(feat_pad, dego2, degi2)


def _tc_dense(aggp, nd_col, W, b_row, ns_col=None):
    def body(a_ref, nd_ref, w_ref, b_ref, *rest):
        if ns_col is not None:
            ns_ref, o_ref = rest
        else:
            (o_ref,) = rest
        a = jnp.concatenate([a_ref[0], a_ref[1]], axis=1) * nd_ref[...]
        y = jnp.dot(a, w_ref[...], preferred_element_type=jnp.float32)
        y = y + b_ref[...]
        if ns_col is not None:
            y = jnp.maximum(y, 0.0) * ns_ref[...]
        o_ref[...] = y

    in_specs = [
        pl.BlockSpec((NC, _BM, DH), lambda i: (0, i, 0)),
        pl.BlockSpec((_BM, 1), lambda i: (i, 0)),
        pl.BlockSpec((D, D), lambda i: (0, 0)),
        pl.BlockSpec((1, D), lambda i: (0, 0)),
    ]
    args = [aggp, nd_col, W, b_row]
    if ns_col is not None:
        in_specs.append(pl.BlockSpec((_BM, 1), lambda i: (i, 0)))
        args.append(ns_col)
    return pl.pallas_call(
        body,
        grid=(NN // _BM,),
        in_specs=in_specs,
        out_specs=pl.BlockSpec((_BM, D), lambda i: (i, 0)),
        out_shape=jax.ShapeDtypeStruct((NN, D), jnp.float32),
    )(*args)


def kernel(feat_data, edge_index, W1, b1, W2, b2):
    n_edges = edge_index.shape[1]
    src = edge_index[0].astype(jnp.int32)
    dst = edge_index[1].astype(jnp.int32)
    src2d, dst2d, nch_tot = _pad_edges(src, dst, n_edges)

    feat_pad = jnp.pad(feat_data, ((0, NN - N_NODES), (0, 0)))
    zeros1 = jnp.zeros((NN,), jnp.float32)
    zeros2 = jnp.zeros((RPT, DH), jnp.float32)
    b1_row = b1.reshape(1, D)
    b2_row = b2.reshape(1, D)

    degp = _sc_degrees(src2d, dst2d, zeros1, nch_tot)
    dego2 = degp[:, 0, :].reshape(NC, NN, 1)
    degi2 = degp[:, 1, :].reshape(NC, NN, 1)

    h1, ns_col, nd_col = _tc_prep(feat_pad, dego2, degi2)
    aggp1 = _sc_aggregate(h1[:, :DH], h1[:, DH:], src2d, dst2d, zeros2,
                          nch_tot)
    h2 = _tc_dense(aggp1, nd_col, W1, b1_row, ns_col=ns_col)
    aggp2 = _sc_aggregate(h2[:, :DH], h2[:, DH:], src2d, dst2d, zeros2,
                          nch_tot)
    out = _tc_dense(aggp2, nd_col, W2, b2_row)
    return out[:N_NODES]


# speedup vs baseline: 11.1571x; 1.0740x over previous
import functools

import jax
import jax.numpy as jnp
from jax import lax
from jax.experimental import pallas as pl
from jax.experimental.pallas import tpu as pltpu
from jax.experimental.pallas import tpu_sc as plsc

N_NODES = 10000
D = 128
DH = D // 2

NC = 2
NS = 16
NW = NC * NS
C = 128

NN = 10240
RPT = NN // NS


def _pad_edges(src, dst, n_edges):
    epw = ((n_edges + NW - 1) // NW + 8 * C - 1) // (8 * C) * (8 * C)
    ep = epw * NW
    pad = ep - n_edges
    pad_ids = N_NODES + (jnp.arange(pad, dtype=jnp.int32) % (NN - N_NODES))
    src_p = jnp.concatenate([src, pad_ids])
    dst_p = jnp.concatenate([dst, pad_ids])
    return src_p.reshape(ep // C, C), dst_p.reshape(ep // C, C), ep // C


_MESH = plsc.VectorSubcoreMesh(core_axis_name="c", subcore_axis_name="s")


def _sc_degrees(src2d, dst2d, zeros1, nch_tot):
    nch = nch_tot // NW

    @functools.partial(
        pl.kernel,
        out_type=jax.ShapeDtypeStruct((NC, 2, NN), jnp.float32),
        mesh=_MESH,
        scratch_types=[
            pltpu.VMEM((nch, C), jnp.int32),
            pltpu.VMEM((nch, C), jnp.int32),
            pltpu.VMEM((C,), jnp.float32),
            pltpu.VMEM_SHARED((NN,), jnp.float32),
            pltpu.VMEM_SHARED((NN,), jnp.float32),
        ],
    )
    def deg_kernel(src_hbm, dst_hbm, z_hbm, out_hbm,
                   src_v, dst_v, ones_v, dego_sp, degi_sp):
        c = lax.axis_index("c")
        s = lax.axis_index("s")
        w = c * NS + s
        pltpu.sync_copy(z_hbm.at[pl.ds(0, RPT)], dego_sp.at[pl.ds(s * RPT, RPT)])
        pltpu.sync_copy(z_hbm.at[pl.ds(0, RPT)], degi_sp.at[pl.ds(s * RPT, RPT)])
        pltpu.sync_copy(src_hbm.at[pl.ds(w * nch, nch)], src_v)
        pltpu.sync_copy(dst_hbm.at[pl.ds(w * nch, nch)], dst_v)
        for i in range(C // 16):
            ones_v[pl.ds(16 * i, 16)] = jnp.full((16,), 1.0, jnp.float32)
        plsc.subcore_barrier()

        def body(g, carry):
            pltpu.sync_copy(ones_v, dego_sp.at[src_v.at[g]], add=True)
            pltpu.sync_copy(ones_v, degi_sp.at[dst_v.at[g]], add=True)
            return carry

        lax.fori_loop(0, nch, body, 0)
        plsc.subcore_barrier()
        pltpu.sync_copy(dego_sp.at[pl.ds(s * RPT, RPT)],
                        out_hbm.at[c, 0, pl.ds(s * RPT, RPT)])
        pltpu.sync_copy(degi_sp.at[pl.ds(s * RPT, RPT)],
                        out_hbm.at[c, 1, pl.ds(s * RPT, RPT)])

    return deg_kernel(src2d, dst2d, zeros1)


_NBUF = 5


def _sc_aggregate(h_lo, h_hi, src2d, dst2d, zeros2, nch_tot):
    nch = nch_tot // NS

    @functools.partial(
        pl.kernel,
        out_type=jax.ShapeDtypeStruct((NC, NN, DH), jnp.float32),
        mesh=_MESH,
        compiler_params=pltpu.CompilerParams(use_tc_tiling_on_sc=False),
        scratch_types=[
            pltpu.VMEM((nch, C), jnp.int32),
            pltpu.VMEM((nch, C), jnp.int32),
            pltpu.VMEM((_NBUF, C, DH), jnp.float32),
            pltpu.VMEM_SHARED((NN, DH), jnp.float32),
            [pltpu.SemaphoreType.DMA] * _NBUF,
            [pltpu.SemaphoreType.DMA] * _NBUF,
        ],
    )
    def agg_kernel(hlo_hbm, hhi_hbm, src_hbm, dst_hbm, z_hbm, out_hbm,
                   src_v, dst_v, rows, agg_sp, sg, ss):
        c = lax.axis_index("c")
        s = lax.axis_index("s")
        pltpu.sync_copy(z_hbm, agg_sp.at[pl.ds(s * RPT, RPT)])
        pltpu.sync_copy(src_hbm.at[pl.ds(s * nch, nch)], src_v)
        pltpu.sync_copy(dst_hbm.at[pl.ds(s * nch, nch)], dst_v)
        plsc.subcore_barrier()

        def one_core(h_hbm):
            def gather(b, g):
                pltpu.async_copy(h_hbm.at[src_v.at[g]], rows.at[b], sg[b])

            def gather_wait(b, g):
                pltpu.make_async_copy(h_hbm.at[src_v.at[g]], rows.at[b],
                                      sg[b]).wait()

            def scatter(b, g):
                pltpu.async_copy(rows.at[b], agg_sp.at[dst_v.at[g]],
                                 ss[b], add=True)

            def scatter_wait(b, g):
                pltpu.make_async_copy(rows.at[b], agg_sp.at[dst_v.at[g]],
                                      ss[b]).wait()

            for b in range(_NBUF):
                gather(b, b)

            def body(k, carry):
                g0 = _NBUF * k
                for b in range(_NBUF):
                    gather_wait(b, g0 + b)
                    scatter(b, g0 + b)
                for b in range(_NBUF):
                    scatter_wait(b, g0 + b)
                    gather(b, g0 + _NBUF + b)
                return carry

            lax.fori_loop(0, nch // _NBUF - 1, body, 0)
            g0 = nch - _NBUF
            for b in range(_NBUF):
                gather_wait(b, g0 + b)
                scatter(b, g0 + b)
            for b in range(_NBUF):
                scatter_wait(b, g0 + b)

        @pl.when(c == 0)
        def _():
            one_core(hlo_hbm)

        @pl.when(c == 1)
        def _():
            one_core(hhi_hbm)

        plsc.subcore_barrier()
        pltpu.sync_copy(agg_sp.at[pl.ds(s * RPT, RPT)],
                        out_hbm.at[c, pl.ds(s * RPT, RPT)])

    return agg_kernel(h_lo, h_hi, src2d, dst2d, zeros2)


_BM = 1024


def _tc_prep(feat, degp):
    def body(f_ref, dg_ref, h1_ref, ns_ref, nd_ref):
        do = dg_ref[0, 0] + dg_ref[1, 0]
        di = dg_ref[0, 1] + dg_ref[1, 1]
        ns = lax.rsqrt(jnp.maximum(do, 1.0))
        nd = lax.rsqrt(jnp.maximum(di, 1.0))
        h1_ref[...] = f_ref[...] * ns[:, None]
        ns_ref[...] = ns
        nd_ref[...] = nd

    return pl.pallas_call(
        body,
        grid=(NN // _BM,),
        in_specs=[
            pl.BlockSpec((_BM, D), lambda i: (i, 0)),
            pl.BlockSpec((NC, 2, _BM), lambda i: (0, 0, i)),
        ],
        out_specs=[
            pl.BlockSpec((_BM, D), lambda i: (i, 0)),
            pl.BlockSpec((_BM,), lambda i: (i,)),
            pl.BlockSpec((_BM,), lambda i: (i,)),
        ],
        out_shape=[
            jax.ShapeDtypeStruct((NN, D), jnp.float32),
            jax.ShapeDtypeStruct((NN,), jnp.float32),
            jax.ShapeDtypeStruct((NN,), jnp.float32),
        ],
    )(feat, degp)


def _tc_dense(aggp, nd, W, b_row, ns=None):
    def body(a_ref, nd_ref, w_ref, b_ref, *rest):
        if ns is not None:
            ns_ref, o_ref = rest
        else:
            (o_ref,) = rest
        a = jnp.concatenate([a_ref[0], a_ref[1]], axis=1)
        a = a * nd_ref[...][:, None]
        y = jnp.dot(a, w_ref[...], preferred_element_type=jnp.float32)
        y = y + b_ref[...]
        if ns is not None:
            y = jnp.maximum(y, 0.0) * ns_ref[...][:, None]
        o_ref[...] = y

    in_specs = [
        pl.BlockSpec((NC, _BM, DH), lambda i: (0, i, 0)),
        pl.BlockSpec((_BM,), lambda i: (i,)),
        pl.BlockSpec((D, D), lambda i: (0, 0)),
        pl.BlockSpec((1, D), lambda i: (0, 0)),
    ]
    args = [aggp, nd, W, b_row]
    if ns is not None:
        in_specs.append(pl.BlockSpec((_BM,), lambda i: (i,)))
        args.append(ns)
    return pl.pallas_call(
        body,
        grid=(NN // _BM,),
        in_specs=in_specs,
        out_specs=pl.BlockSpec((_BM, D), lambda i: (i, 0)),
        out_shape=jax.ShapeDtypeStruct((NN, D), jnp.float32),
    )(*args)


def kernel(feat_data, edge_index, W1, b1, W2, b2):
    n_edges = edge_index.shape[1]
    src = edge_index[0].astype(jnp.int32)
    dst = edge_index[1].astype(jnp.int32)
    src2d, dst2d, nch_tot = _pad_edges(src, dst, n_edges)

    zeros1 = jnp.zeros((NN,), jnp.float32)
    zeros2 = jnp.zeros((RPT, DH), jnp.float32)
    b1_row = b1.reshape(1, D)
    b2_row = b2.reshape(1, D)

    degp = _sc_degrees(src2d, dst2d, zeros1, nch_tot)

    feat_pad = jnp.pad(feat_data, ((0, NN - N_NODES), (0, 0)))
    h1, ns, nd = _tc_prep(feat_pad, degp)
    aggp1 = _sc_aggregate(h1[:, :DH], h1[:, DH:], src2d, dst2d, zeros2,
                          nch_tot)
    h2 = _tc_dense(aggp1, nd, W1, b1_row, ns=ns)
    aggp2 = _sc_aggregate(h2[:, :DH], h2[:, DH:], src2d, dst2d, zeros2,
                          nch_tot)
    return _tc_dense(aggp2, nd, W2, b2_row)[:N_NODES]


# speedup vs baseline: 11.4953x; 1.0303x over previous
import functools

import jax
import jax.numpy as jnp
from jax import lax
from jax.experimental import pallas as pl
from jax.experimental.pallas import tpu as pltpu
from jax.experimental.pallas import tpu_sc as plsc

N_NODES = 10000
D = 128
DH = D // 2

NC = 2
NS = 16
NW = NC * NS
C = 128

NN = 10240
RPT = NN // NS


_MESH = plsc.VectorSubcoreMesh(core_axis_name="c", subcore_axis_name="s")


def _sc_degrees(edges, zeros1, n_edges):
    nch_tot = n_edges // C
    nch_lo = nch_tot // NW
    n_hi = nch_tot - nch_lo * NW
    nch_hi = nch_lo + 1

    @functools.partial(
        pl.kernel,
        out_type=jax.ShapeDtypeStruct((NC, 2, NN), jnp.float32),
        mesh=_MESH,
        compiler_params=pltpu.CompilerParams(use_tc_tiling_on_sc=False),
        scratch_types=[
            pltpu.VMEM((nch_hi * C,), jnp.int32),
            pltpu.VMEM((nch_hi * C,), jnp.int32),
            pltpu.VMEM((C,), jnp.float32),
            pltpu.VMEM_SHARED((NN,), jnp.float32),
            pltpu.VMEM_SHARED((NN,), jnp.float32),
        ],
    )
    def deg_kernel(e_hbm, z_hbm, out_hbm,
                   src_v, dst_v, ones_v, dego_sp, degi_sp):
        c = lax.axis_index("c")
        s = lax.axis_index("s")
        w = c * NS + s
        pltpu.sync_copy(z_hbm.at[pl.ds(0, RPT)], dego_sp.at[pl.ds(s * RPT, RPT)])
        pltpu.sync_copy(z_hbm.at[pl.ds(0, RPT)], degi_sp.at[pl.ds(s * RPT, RPT)])
        for i in range(C // 16):
            ones_v[pl.ds(16 * i, 16)] = jnp.full((16,), 1.0, jnp.float32)

        def stage(nch, start):
            ne = nch * C
            pltpu.sync_copy(e_hbm.at[0, pl.ds(start, ne)], src_v.at[pl.ds(0, ne)])
            pltpu.sync_copy(e_hbm.at[1, pl.ds(start, ne)], dst_v.at[pl.ds(0, ne)])

        def run(nch):
            def body(g, carry):
                pltpu.sync_copy(ones_v, dego_sp.at[src_v.at[pl.ds(g * C, C)]],
                                add=True)
                pltpu.sync_copy(ones_v, degi_sp.at[dst_v.at[pl.ds(g * C, C)]],
                                add=True)
                return carry

            lax.fori_loop(0, nch, body, 0)

        hi = w < n_hi
        pl.when(hi)(lambda: stage(nch_hi, w * nch_hi * C))
        pl.when(~hi)(lambda: stage(nch_lo, (n_hi + w * nch_lo) * C))
        plsc.subcore_barrier()
        pl.when(hi)(lambda: run(nch_hi))
        pl.when(~hi)(lambda: run(nch_lo))
        plsc.subcore_barrier()
        pltpu.sync_copy(dego_sp.at[pl.ds(s * RPT, RPT)],
                        out_hbm.at[c, 0, pl.ds(s * RPT, RPT)])
        pltpu.sync_copy(degi_sp.at[pl.ds(s * RPT, RPT)],
                        out_hbm.at[c, 1, pl.ds(s * RPT, RPT)])

    return deg_kernel(edges, zeros1)


_NBUF = 5


def _sc_aggregate(h_lo, h_hi, edges, zeros2, n_edges):
    nch_tot = n_edges // C
    nch_lo = nch_tot // NS // _NBUF * _NBUF
    n_hi = (nch_tot - nch_lo * NS) // _NBUF
    nch_hi = nch_lo + _NBUF
    assert nch_hi * n_hi + nch_lo * (NS - n_hi) == nch_tot

    @functools.partial(
        pl.kernel,
        out_type=jax.ShapeDtypeStruct((NC, NN, DH), jnp.float32),
        mesh=_MESH,
        compiler_params=pltpu.CompilerParams(use_tc_tiling_on_sc=False),
        scratch_types=[
            pltpu.VMEM((nch_hi * C,), jnp.int32),
            pltpu.VMEM((nch_hi * C,), jnp.int32),
            pltpu.VMEM((_NBUF, C, DH), jnp.float32),
            pltpu.VMEM_SHARED((NN, DH), jnp.float32),
            [pltpu.SemaphoreType.DMA] * _NBUF,
            [pltpu.SemaphoreType.DMA] * _NBUF,
        ],
    )
    def agg_kernel(hlo_hbm, hhi_hbm, e_hbm, z_hbm, out_hbm,
                   src_v, dst_v, rows, agg_sp, sg, ss):
        c = lax.axis_index("c")
        s = lax.axis_index("s")
        pltpu.sync_copy(z_hbm, agg_sp.at[pl.ds(s * RPT, RPT)])

        def stage(nch, start):
            ne = nch * C
            pltpu.sync_copy(e_hbm.at[0, pl.ds(start, ne)], src_v.at[pl.ds(0, ne)])
            pltpu.sync_copy(e_hbm.at[1, pl.ds(start, ne)], dst_v.at[pl.ds(0, ne)])

        def run(h_hbm, nch):
            def gather(b, g):
                pltpu.async_copy(h_hbm.at[src_v.at[pl.ds(g * C, C)]],
                                 rows.at[b], sg[b])

            def gather_wait(b, g):
                pltpu.make_async_copy(h_hbm.at[src_v.at[pl.ds(g * C, C)]],
                                      rows.at[b], sg[b]).wait()

            def scatter(b, g):
                pltpu.async_copy(rows.at[b], agg_sp.at[dst_v.at[pl.ds(g * C, C)]],
                                 ss[b], add=True)

            def scatter_wait(b, g):
                pltpu.make_async_copy(rows.at[b],
                                      agg_sp.at[dst_v.at[pl.ds(g * C, C)]],
                                      ss[b]).wait()

            for b in range(_NBUF):
                gather(b, b)

            def body(k, carry):
                g0 = _NBUF * k
                for b in range(_NBUF):
                    gather_wait(b, g0 + b)
                    scatter(b, g0 + b)
                for b in range(_NBUF):
                    scatter_wait(b, g0 + b)
                    gather(b, g0 + _NBUF + b)
                return carry

            lax.fori_loop(0, nch // _NBUF - 1, body, 0)
            g0 = nch - _NBUF
            for b in range(_NBUF):
                gather_wait(b, g0 + b)
                scatter(b, g0 + b)
            for b in range(_NBUF):
                scatter_wait(b, g0 + b)

        hi = s < n_hi
        pl.when(hi)(lambda: stage(nch_hi, s * nch_hi * C))
        pl.when(~hi)(lambda: stage(nch_lo, (n_hi * _NBUF + s * nch_lo) * C))
        plsc.subcore_barrier()

        def run_core(nch):
            pl.when(c == 0)(lambda: run(hlo_hbm, nch))
            pl.when(c == 1)(lambda: run(hhi_hbm, nch))

        pl.when(hi)(lambda: run_core(nch_hi))
        pl.when(~hi)(lambda: run_core(nch_lo))

        plsc.subcore_barrier()
        pltpu.sync_copy(agg_sp.at[pl.ds(s * RPT, RPT)],
                        out_hbm.at[c, pl.ds(s * RPT, RPT)])

    return agg_kernel(h_lo, h_hi, edges, zeros2)


_BM = 1024


def _tc_prep(feat, degp):
    def body(f_ref, dg_ref, h1_ref, ns_ref, nd_ref):
        do = dg_ref[0, 0] + dg_ref[1, 0]
        di = dg_ref[0, 1] + dg_ref[1, 1]
        ns = lax.rsqrt(jnp.maximum(do, 1.0))
        nd = lax.rsqrt(jnp.maximum(di, 1.0))
        h1_ref[...] = f_ref[...] * ns[:, None]
        ns_ref[...] = ns
        nd_ref[...] = nd

    return pl.pallas_call(
        body,
        grid=(NN // _BM,),
        in_specs=[
            pl.BlockSpec((_BM, D), lambda i: (i, 0)),
            pl.BlockSpec((NC, 2, _BM), lambda i: (0, 0, i)),
        ],
        out_specs=[
            pl.BlockSpec((_BM, D), lambda i: (i, 0)),
            pl.BlockSpec((_BM,), lambda i: (i,)),
            pl.BlockSpec((_BM,), lambda i: (i,)),
        ],
        out_shape=[
            jax.ShapeDtypeStruct((NN, D), jnp.float32),
            jax.ShapeDtypeStruct((NN,), jnp.float32),
            jax.ShapeDtypeStruct((NN,), jnp.float32),
        ],
    )(feat, degp)


def _tc_dense(aggp, nd, W, b_row, ns=None):
    def body(a_ref, nd_ref, w_ref, b_ref, *rest):
        if ns is not None:
            ns_ref, o_ref = rest
        else:
            (o_ref,) = rest
        a = jnp.concatenate([a_ref[0], a_ref[1]], axis=1)
        a = a * nd_ref[...][:, None]
        y = jnp.dot(a, w_ref[...], preferred_element_type=jnp.float32)
        y = y + b_ref[...]
        if ns is not None:
            y = jnp.maximum(y, 0.0) * ns_ref[...][:, None]
        o_ref[...] = y

    in_specs = [
        pl.BlockSpec((NC, _BM, DH), lambda i: (0, i, 0)),
        pl.BlockSpec((_BM,), lambda i: (i,)),
        pl.BlockSpec((D, D), lambda i: (0, 0)),
        pl.BlockSpec((1, D), lambda i: (0, 0)),
    ]
    args = [aggp, nd, W, b_row]
    if ns is not None:
        in_specs.append(pl.BlockSpec((_BM,), lambda i: (i,)))
        args.append(ns)
    return pl.pallas_call(
        body,
        grid=(NN // _BM,),
        in_specs=in_specs,
        out_specs=pl.BlockSpec((_BM, D), lambda i: (i, 0)),
        out_shape=jax.ShapeDtypeStruct((NN, D), jnp.float32),
    )(*args)


def kernel(feat_data, edge_index, W1, b1, W2, b2):
    n_edges = edge_index.shape[1]
    edges = edge_index.astype(jnp.int32)

    zeros1 = jnp.zeros((NN,), jnp.float32)
    zeros2 = jnp.zeros((RPT, DH), jnp.float32)
    b1_row = b1.reshape(1, D)
    b2_row = b2.reshape(1, D)

    degp = _sc_degrees(edges, zeros1, n_edges)

    feat_pad = jnp.pad(feat_data, ((0, NN - N_NODES), (0, 0)))
    h1, ns, nd = _tc_prep(feat_pad, degp)
    aggp1 = _sc_aggregate(h1[:, :DH], h1[:, DH:], edges, zeros2,
                          n_edges)
    h2 = _tc_dense(aggp1, nd, W1, b1_row, ns=ns)
    aggp2 = _sc_aggregate(h2[:, :DH], h2[:, DH:], edges, zeros2, n_edges)
    return _tc_dense(aggp2, nd, W2, b2_row)[:N_NODES]


# speedup vs baseline: 11.8388x; 1.0299x over previous
import functools

import jax
import jax.numpy as jnp
from jax import lax
from jax.experimental import pallas as pl
from jax.experimental.pallas import tpu as pltpu
from jax.experimental.pallas import tpu_sc as plsc

N_NODES = 10000
D = 128
DH = D // 2

NC = 2
NS = 16
NW = NC * NS
C = 128

NN = 10240
RPT = NN // NS


_MESH = plsc.VectorSubcoreMesh(core_axis_name="c", subcore_axis_name="s")


def _sc_degrees(edges, zeros1, n_edges):
    nch_tot = n_edges // C
    nch_lo = nch_tot // NW
    n_hi = nch_tot - nch_lo * NW
    nch_hi = nch_lo + 1

    @functools.partial(
        pl.kernel,
        out_type=jax.ShapeDtypeStruct((NC, 2, NN), jnp.float32),
        mesh=_MESH,
        compiler_params=pltpu.CompilerParams(use_tc_tiling_on_sc=False),
        scratch_types=[
            pltpu.VMEM((nch_hi * C,), jnp.int32),
            pltpu.VMEM((nch_hi * C,), jnp.int32),
            pltpu.VMEM((C,), jnp.float32),
            pltpu.VMEM_SHARED((NN,), jnp.float32),
            pltpu.VMEM_SHARED((NN,), jnp.float32),
            pltpu.SemaphoreType.DMA,
            pltpu.SemaphoreType.DMA,
        ],
    )
    def deg_kernel(e_hbm, z_hbm, out_hbm,
                   src_v, dst_v, ones_v, dego_sp, degi_sp, sem_s, sem_d):
        c = lax.axis_index("c")
        s = lax.axis_index("s")
        w = c * NS + s
        pltpu.sync_copy(z_hbm.at[pl.ds(0, RPT)], dego_sp.at[pl.ds(s * RPT, RPT)])
        pltpu.sync_copy(z_hbm.at[pl.ds(0, RPT)], degi_sp.at[pl.ds(s * RPT, RPT)])
        for i in range(C // 16):
            ones_v[pl.ds(16 * i, 16)] = jnp.full((16,), 1.0, jnp.float32)

        def stage(nch, start):
            ne = nch * C
            pltpu.sync_copy(e_hbm.at[0, pl.ds(start, ne)], src_v.at[pl.ds(0, ne)])
            pltpu.sync_copy(e_hbm.at[1, pl.ds(start, ne)], dst_v.at[pl.ds(0, ne)])

        def run(nch):
            _GRP = 6

            def s_scat(g):
                return (ones_v, dego_sp.at[src_v.at[pl.ds(g * C, C)]], sem_s)

            def d_scat(g):
                return (ones_v, degi_sp.at[dst_v.at[pl.ds(g * C, C)]], sem_d)

            def body(k, carry):
                g0 = _GRP * k
                for j in range(_GRP):
                    pltpu.async_copy(*s_scat(g0 + j), add=True)
                    pltpu.async_copy(*d_scat(g0 + j), add=True)
                for j in range(_GRP):
                    pltpu.make_async_copy(*s_scat(g0 + j)).wait()
                    pltpu.make_async_copy(*d_scat(g0 + j)).wait()
                return carry

            lax.fori_loop(0, nch // _GRP, body, 0)
            for g in range(nch // _GRP * _GRP, nch):
                pltpu.sync_copy(ones_v, dego_sp.at[src_v.at[pl.ds(g * C, C)]],
                                add=True)
                pltpu.sync_copy(ones_v, degi_sp.at[dst_v.at[pl.ds(g * C, C)]],
                                add=True)

        hi = w < n_hi
        pl.when(hi)(lambda: stage(nch_hi, w * nch_hi * C))
        pl.when(~hi)(lambda: stage(nch_lo, (n_hi + w * nch_lo) * C))
        plsc.subcore_barrier()
        pl.when(hi)(lambda: run(nch_hi))
        pl.when(~hi)(lambda: run(nch_lo))
        plsc.subcore_barrier()
        pltpu.sync_copy(dego_sp.at[pl.ds(s * RPT, RPT)],
                        out_hbm.at[c, 0, pl.ds(s * RPT, RPT)])
        pltpu.sync_copy(degi_sp.at[pl.ds(s * RPT, RPT)],
                        out_hbm.at[c, 1, pl.ds(s * RPT, RPT)])

    return deg_kernel(edges, zeros1)


_NBUF = 5


def _sc_aggregate(h_lo, h_hi, edges, zeros2, n_edges):
    nch_tot = n_edges // C
    nch_lo = nch_tot // NS // _NBUF * _NBUF
    n_hi = (nch_tot - nch_lo * NS) // _NBUF
    nch_hi = nch_lo + _NBUF
    assert nch_hi * n_hi + nch_lo * (NS - n_hi) == nch_tot

    @functools.partial(
        pl.kernel,
        out_type=jax.ShapeDtypeStruct((NC, NN, DH), jnp.float32),
        mesh=_MESH,
        compiler_params=pltpu.CompilerParams(use_tc_tiling_on_sc=False),
        scratch_types=[
            pltpu.VMEM((nch_hi * C,), jnp.int32),
            pltpu.VMEM((nch_hi * C,), jnp.int32),
            pltpu.VMEM((_NBUF, C, DH), jnp.float32),
            pltpu.VMEM_SHARED((NN, DH), jnp.float32),
            [pltpu.SemaphoreType.DMA] * _NBUF,
            [pltpu.SemaphoreType.DMA] * _NBUF,
        ],
    )
    def agg_kernel(hlo_hbm, hhi_hbm, e_hbm, z_hbm, out_hbm,
                   src_v, dst_v, rows, agg_sp, sg, ss):
        c = lax.axis_index("c")
        s = lax.axis_index("s")
        pltpu.sync_copy(z_hbm, agg_sp.at[pl.ds(s * RPT, RPT)])

        def stage(nch, start):
            ne = nch * C
            pltpu.sync_copy(e_hbm.at[0, pl.ds(start, ne)], src_v.at[pl.ds(0, ne)])
            pltpu.sync_copy(e_hbm.at[1, pl.ds(start, ne)], dst_v.at[pl.ds(0, ne)])

        def run(h_hbm, nch):
            def gather(b, g):
                pltpu.async_copy(h_hbm.at[src_v.at[pl.ds(g * C, C)]],
                                 rows.at[b], sg[b])

            def gather_wait(b, g):
                pltpu.make_async_copy(h_hbm.at[src_v.at[pl.ds(g * C, C)]],
                                      rows.at[b], sg[b]).wait()

            def scatter(b, g):
                pltpu.async_copy(rows.at[b], agg_sp.at[dst_v.at[pl.ds(g * C, C)]],
                                 ss[b], add=True)

            def scatter_wait(b, g):
                pltpu.make_async_copy(rows.at[b],
                                      agg_sp.at[dst_v.at[pl.ds(g * C, C)]],
                                      ss[b]).wait()

            for b in range(_NBUF):
                gather(b, b)

            def body(k, carry):
                g0 = _NBUF * k
                for b in range(_NBUF):
                    gather_wait(b, g0 + b)
                    scatter(b, g0 + b)
                for b in range(_NBUF):
                    scatter_wait(b, g0 + b)
                    gather(b, g0 + _NBUF + b)
                return carry

            lax.fori_loop(0, nch // _NBUF - 1, body, 0)
            g0 = nch - _NBUF
            for b in range(_NBUF):
                gather_wait(b, g0 + b)
                scatter(b, g0 + b)
            for b in range(_NBUF):
                scatter_wait(b, g0 + b)

        hi = s < n_hi
        pl.when(hi)(lambda: stage(nch_hi, s * nch_hi * C))
        pl.when(~hi)(lambda: stage(nch_lo, (n_hi * _NBUF + s * nch_lo) * C))
        plsc.subcore_barrier()

        def run_core(nch):
            pl.when(c == 0)(lambda: run(hlo_hbm, nch))
            pl.when(c == 1)(lambda: run(hhi_hbm, nch))

        pl.when(hi)(lambda: run_core(nch_hi))
        pl.when(~hi)(lambda: run_core(nch_lo))

        plsc.subcore_barrier()
        pltpu.sync_copy(agg_sp.at[pl.ds(s * RPT, RPT)],
                        out_hbm.at[c, pl.ds(s * RPT, RPT)])

    return agg_kernel(h_lo, h_hi, edges, zeros2)


_BM = 1024


def _tc_prep(feat, degp):
    def body(f_ref, dg_ref, h1_ref, ns_ref, nd_ref):
        do = dg_ref[0, 0] + dg_ref[1, 0]
        di = dg_ref[0, 1] + dg_ref[1, 1]
        ns = lax.rsqrt(jnp.maximum(do, 1.0))
        nd = lax.rsqrt(jnp.maximum(di, 1.0))
        h1_ref[...] = f_ref[...] * ns[:, None]
        ns_ref[...] = ns
        nd_ref[...] = nd

    return pl.pallas_call(
        body,
        grid=(NN // _BM,),
        in_specs=[
            pl.BlockSpec((_BM, D), lambda i: (i, 0)),
            pl.BlockSpec((NC, 2, _BM), lambda i: (0, 0, i)),
        ],
        out_specs=[
            pl.BlockSpec((_BM, D), lambda i: (i, 0)),
            pl.BlockSpec((_BM,), lambda i: (i,)),
            pl.BlockSpec((_BM,), lambda i: (i,)),
        ],
        out_shape=[
            jax.ShapeDtypeStruct((NN, D), jnp.float32),
            jax.ShapeDtypeStruct((NN,), jnp.float32),
            jax.ShapeDtypeStruct((NN,), jnp.float32),
        ],
    )(feat, degp)


def _tc_dense(aggp, nd, W, b_row, ns=None):
    def body(a_ref, nd_ref, w_ref, b_ref, *rest):
        if ns is not None:
            ns_ref, o_ref = rest
        else:
            (o_ref,) = rest
        ndc = nd_ref[...][:, None]
        y = jnp.dot(a_ref[0] * ndc, w_ref[:DH, :],
                    preferred_element_type=jnp.float32)
        y = y + jnp.dot(a_ref[1] * ndc, w_ref[DH:, :],
                        preferred_element_type=jnp.float32)
        y = y + b_ref[...]
        if ns is not None:
            y = jnp.maximum(y, 0.0) * ns_ref[...][:, None]
        o_ref[...] = y

    in_specs = [
        pl.BlockSpec((NC, _BM, DH), lambda i: (0, i, 0)),
        pl.BlockSpec((_BM,), lambda i: (i,)),
        pl.BlockSpec((D, D), lambda i: (0, 0)),
        pl.BlockSpec((1, D), lambda i: (0, 0)),
    ]
    args = [aggp, nd, W, b_row]
    if ns is not None:
        in_specs.append(pl.BlockSpec((_BM,), lambda i: (i,)))
        args.append(ns)
    return pl.pallas_call(
        body,
        grid=(NN // _BM,),
        in_specs=in_specs,
        out_specs=pl.BlockSpec((_BM, D), lambda i: (i, 0)),
        out_shape=jax.ShapeDtypeStruct((NN, D), jnp.float32),
    )(*args)


def kernel(feat_data, edge_index, W1, b1, W2, b2):
    n_edges = edge_index.shape[1]
    edges = edge_index.astype(jnp.int32)

    zeros1 = jnp.zeros((NN,), jnp.float32)
    zeros2 = jnp.zeros((RPT, DH), jnp.float32)
    b1_row = b1.reshape(1, D)
    b2_row = b2.reshape(1, D)

    degp = _sc_degrees(edges, zeros1, n_edges)

    feat_pad = jnp.pad(feat_data, ((0, NN - N_NODES), (0, 0)))
    h1, ns, nd = _tc_prep(feat_pad, degp)
    aggp1 = _sc_aggregate(h1[:, :DH], h1[:, DH:], edges, zeros2,
                          n_edges)
    h2 = _tc_dense(aggp1, nd, W1, b1_row, ns=ns)
    aggp2 = _sc_aggregate(h2[:, :DH], h2[:, DH:], edges, zeros2, n_edges)
    return _tc_dense(aggp2, nd, W2, b2_row)[:N_NODES]


# speedup vs baseline: 12.5663x; 1.0615x over previous
import functools

import jax
import jax.numpy as jnp
from jax import lax
from jax.experimental import pallas as pl
from jax.experimental.pallas import tpu as pltpu
from jax.experimental.pallas import tpu_sc as plsc

N_NODES = 10000
D = 128
DH = D // 2

NC = 2
NS = 16
NW = NC * NS
C = 128

NN = 10240
RPT = NN // NS


_MESH = plsc.VectorSubcoreMesh(core_axis_name="c", subcore_axis_name="s")


def _sc_degrees(edges, zeros1, n_edges):
    nch_tot = n_edges // C
    nch_lo = nch_tot // NW
    n_hi = nch_tot - nch_lo * NW
    nch_hi = nch_lo + 1

    @functools.partial(
        pl.kernel,
        out_type=jax.ShapeDtypeStruct((NC, 2, NN), jnp.float32),
        mesh=_MESH,
        compiler_params=pltpu.CompilerParams(use_tc_tiling_on_sc=False),
        scratch_types=[
            pltpu.VMEM((nch_hi * C,), jnp.int32),
            pltpu.VMEM((nch_hi * C,), jnp.int32),
            pltpu.VMEM((C,), jnp.float32),
            pltpu.VMEM_SHARED((NN,), jnp.float32),
            pltpu.VMEM_SHARED((NN,), jnp.float32),
            pltpu.SemaphoreType.DMA,
            pltpu.SemaphoreType.DMA,
        ],
    )
    def deg_kernel(e_hbm, z_hbm, out_hbm,
                   src_v, dst_v, ones_v, dego_sp, degi_sp, sem_s, sem_d):
        c = lax.axis_index("c")
        s = lax.axis_index("s")
        w = c * NS + s
        pltpu.sync_copy(z_hbm.at[pl.ds(0, RPT)], dego_sp.at[pl.ds(s * RPT, RPT)])
        pltpu.sync_copy(z_hbm.at[pl.ds(0, RPT)], degi_sp.at[pl.ds(s * RPT, RPT)])
        for i in range(C // 16):
            ones_v[pl.ds(16 * i, 16)] = jnp.full((16,), 1.0, jnp.float32)

        def stage(nch, start):
            ne = nch * C
            pltpu.sync_copy(e_hbm.at[0, pl.ds(start, ne)], src_v.at[pl.ds(0, ne)])
            pltpu.sync_copy(e_hbm.at[1, pl.ds(start, ne)], dst_v.at[pl.ds(0, ne)])

        def run(nch):
            _GRP = 6

            def s_scat(g):
                return (ones_v, dego_sp.at[src_v.at[pl.ds(g * C, C)]], sem_s)

            def d_scat(g):
                return (ones_v, degi_sp.at[dst_v.at[pl.ds(g * C, C)]], sem_d)

            def body(k, carry):
                g0 = _GRP * k
                for j in range(_GRP):
                    pltpu.async_copy(*s_scat(g0 + j), add=True)
                    pltpu.async_copy(*d_scat(g0 + j), add=True)
                for j in range(_GRP):
                    pltpu.make_async_copy(*s_scat(g0 + j)).wait()
                    pltpu.make_async_copy(*d_scat(g0 + j)).wait()
                return carry

            lax.fori_loop(0, nch // _GRP, body, 0)
            for g in range(nch // _GRP * _GRP, nch):
                pltpu.sync_copy(ones_v, dego_sp.at[src_v.at[pl.ds(g * C, C)]],
                                add=True)
                pltpu.sync_copy(ones_v, degi_sp.at[dst_v.at[pl.ds(g * C, C)]],
                                add=True)

        hi = w < n_hi
        pl.when(hi)(lambda: stage(nch_hi, w * nch_hi * C))
        pl.when(~hi)(lambda: stage(nch_lo, (n_hi + w * nch_lo) * C))
        plsc.subcore_barrier()
        pl.when(hi)(lambda: run(nch_hi))
        pl.when(~hi)(lambda: run(nch_lo))
        plsc.subcore_barrier()
        pltpu.sync_copy(dego_sp.at[pl.ds(s * RPT, RPT)],
                        out_hbm.at[c, 0, pl.ds(s * RPT, RPT)])
        pltpu.sync_copy(degi_sp.at[pl.ds(s * RPT, RPT)],
                        out_hbm.at[c, 1, pl.ds(s * RPT, RPT)])

    return deg_kernel(edges, zeros1)


_NBUF = 5


def _sc_aggregate(h_lo, h_hi, edges, zeros2, n_edges):
    nch_tot = n_edges // C
    nch_lo = nch_tot // NS // _NBUF * _NBUF
    n_hi = (nch_tot - nch_lo * NS) // _NBUF
    nch_hi = nch_lo + _NBUF
    assert nch_hi * n_hi + nch_lo * (NS - n_hi) == nch_tot

    @functools.partial(
        pl.kernel,
        out_type=jax.ShapeDtypeStruct((NC, NN, DH), jnp.float32),
        mesh=_MESH,
        compiler_params=pltpu.CompilerParams(use_tc_tiling_on_sc=False),
        scratch_types=[
            pltpu.VMEM((nch_hi * C,), jnp.int32),
            pltpu.VMEM((nch_hi * C,), jnp.int32),
            pltpu.VMEM((_NBUF, C, DH), jnp.float32),
            pltpu.VMEM_SHARED((NN, DH), jnp.float32),
            [pltpu.SemaphoreType.DMA] * _NBUF,
            [pltpu.SemaphoreType.DMA] * _NBUF,
        ],
    )
    def agg_kernel(hlo_hbm, hhi_hbm, e_hbm, z_hbm, out_hbm,
                   src_v, dst_v, rows, agg_sp, sg, ss):
        c = lax.axis_index("c")
        s = lax.axis_index("s")
        pltpu.sync_copy(z_hbm, agg_sp.at[pl.ds(s * RPT, RPT)])

        def stage(nch, start):
            ne = nch * C
            pltpu.sync_copy(e_hbm.at[0, pl.ds(start, ne)], src_v.at[pl.ds(0, ne)])
            pltpu.sync_copy(e_hbm.at[1, pl.ds(start, ne)], dst_v.at[pl.ds(0, ne)])

        def run(h_hbm, nch):
            def gather(b, g):
                pltpu.async_copy(h_hbm.at[src_v.at[pl.ds(g * C, C)]],
                                 rows.at[b], sg[b])

            def gather_wait(b, g):
                pltpu.make_async_copy(h_hbm.at[src_v.at[pl.ds(g * C, C)]],
                                      rows.at[b], sg[b]).wait()

            def scatter(b, g):
                pltpu.async_copy(rows.at[b], agg_sp.at[dst_v.at[pl.ds(g * C, C)]],
                                 ss[b], add=True)

            def scatter_wait(b, g):
                pltpu.make_async_copy(rows.at[b],
                                      agg_sp.at[dst_v.at[pl.ds(g * C, C)]],
                                      ss[b]).wait()

            for b in range(_NBUF):
                gather(b, b)

            def body(k, carry):
                g0 = _NBUF * k
                for b in range(_NBUF):
                    gather_wait(b, g0 + b)
                    scatter(b, g0 + b)
                for b in range(_NBUF):
                    scatter_wait(b, g0 + b)
                    gather(b, g0 + _NBUF + b)
                return carry

            lax.fori_loop(0, nch // _NBUF - 1, body, 0)
            g0 = nch - _NBUF
            for b in range(_NBUF):
                gather_wait(b, g0 + b)
                scatter(b, g0 + b)
            for b in range(_NBUF):
                scatter_wait(b, g0 + b)

        hi = s < n_hi
        pl.when(hi)(lambda: stage(nch_hi, s * nch_hi * C))
        pl.when(~hi)(lambda: stage(nch_lo, (n_hi * _NBUF + s * nch_lo) * C))
        plsc.subcore_barrier()

        def run_core(nch):
            pl.when(c == 0)(lambda: run(hlo_hbm, nch))
            pl.when(c == 1)(lambda: run(hhi_hbm, nch))

        pl.when(hi)(lambda: run_core(nch_hi))
        pl.when(~hi)(lambda: run_core(nch_lo))

        plsc.subcore_barrier()
        pltpu.sync_copy(agg_sp.at[pl.ds(s * RPT, RPT)],
                        out_hbm.at[c, pl.ds(s * RPT, RPT)])

    return agg_kernel(h_lo, h_hi, edges, zeros2)


_BM = 1024
_BP = _BM // 2


def _tc_prep(feat_e, feat_o, degp4):
    def body(fe_ref, fo_ref, dg_ref, lo_ref, hi_ref, ns_ref, nd_ref):
        do = dg_ref[0, 0] + dg_ref[1, 0]
        di = dg_ref[0, 1] + dg_ref[1, 1]
        ns2 = lax.rsqrt(jnp.maximum(do, 1.0))
        nd2 = lax.rsqrt(jnp.maximum(di, 1.0))
        he = fe_ref[...] * ns2[:, 0:1]
        ho = fo_ref[...] * ns2[:, 1:2]
        lo_ref[...] = jnp.concatenate([he[:, :DH], ho[:, :DH]], axis=1)
        hi_ref[...] = jnp.concatenate([he[:, DH:], ho[:, DH:]], axis=1)
        ns_ref[...] = ns2
        nd_ref[...] = nd2

    return pl.pallas_call(
        body,
        grid=(NN // _BM,),
        in_specs=[
            pl.BlockSpec((_BP, D), lambda i: (i, 0)),
            pl.BlockSpec((_BP, D), lambda i: (i, 0)),
            pl.BlockSpec((NC, 2, _BP, 2), lambda i: (0, 0, i, 0)),
        ],
        out_specs=[
            pl.BlockSpec((_BP, D), lambda i: (i, 0)),
            pl.BlockSpec((_BP, D), lambda i: (i, 0)),
            pl.BlockSpec((_BP, 2), lambda i: (i, 0)),
            pl.BlockSpec((_BP, 2), lambda i: (i, 0)),
        ],
        out_shape=[
            jax.ShapeDtypeStruct((NN // 2, D), jnp.float32),
            jax.ShapeDtypeStruct((NN // 2, D), jnp.float32),
            jax.ShapeDtypeStruct((NN // 2, 2), jnp.float32),
            jax.ShapeDtypeStruct((NN // 2, 2), jnp.float32),
        ],
    )(feat_e, feat_o, degp4)


def _tc_dense(aggp, nd2, W, b_row, ns2=None):
    def body(a_ref, nd_ref, w_ref, b_ref, *rest):
        if ns2 is None:
            lo_ref, hi_ref = rest
        else:
            ns_ref, lo_ref, hi_ref = rest
        nde, ndo = nd_ref[:, 0:1], nd_ref[:, 1:2]
        a0, a1 = a_ref[0], a_ref[1]
        b = b_ref[...]
        ye = (jnp.dot(a0[:, :DH] * nde, w_ref[:DH, :],
                      preferred_element_type=jnp.float32)
              + jnp.dot(a1[:, :DH] * nde, w_ref[DH:, :],
                        preferred_element_type=jnp.float32) + b)
        yo = (jnp.dot(a0[:, DH:] * ndo, w_ref[:DH, :],
                      preferred_element_type=jnp.float32)
              + jnp.dot(a1[:, DH:] * ndo, w_ref[DH:, :],
                        preferred_element_type=jnp.float32) + b)
        if ns2 is not None:
            ye = jnp.maximum(ye, 0.0) * ns_ref[:, 0:1]
            yo = jnp.maximum(yo, 0.0) * ns_ref[:, 1:2]
        lo_ref[...] = jnp.concatenate([ye[:, :DH], yo[:, :DH]], axis=1)
        hi_ref[...] = jnp.concatenate([ye[:, DH:], yo[:, DH:]], axis=1)

    in_specs = [
        pl.BlockSpec((NC, _BP, D), lambda i: (0, i, 0)),
        pl.BlockSpec((_BP, 2), lambda i: (i, 0)),
        pl.BlockSpec((D, D), lambda i: (0, 0)),
        pl.BlockSpec((1, D), lambda i: (0, 0)),
    ]
    args = [aggp.reshape(NC, NN // 2, D), nd2, W, b_row]
    if ns2 is not None:
        in_specs.append(pl.BlockSpec((_BP, 2), lambda i: (i, 0)))
        args.append(ns2)
    return pl.pallas_call(
        body,
        grid=(NN // _BM,),
        in_specs=in_specs,
        out_specs=[pl.BlockSpec((_BP, D), lambda i: (i, 0))] * 2,
        out_shape=[jax.ShapeDtypeStruct((NN // 2, D), jnp.float32)] * 2,
    )(*args)


def kernel(feat_data, edge_index, W1, b1, W2, b2):
    n_edges = edge_index.shape[1]
    edges = edge_index.astype(jnp.int32)

    zeros1 = jnp.zeros((NN,), jnp.float32)
    zeros2 = jnp.zeros((RPT, DH), jnp.float32)
    b1_row = b1.reshape(1, D)
    b2_row = b2.reshape(1, D)

    degp = _sc_degrees(edges, zeros1, n_edges)
    degp4 = degp.reshape(NC, 2, NN // 2, 2)

    feat_pad = jnp.pad(feat_data, ((0, NN - N_NODES), (0, 0)))
    h1lo, h1hi, ns2, nd2 = _tc_prep(feat_pad[0::2], feat_pad[1::2], degp4)
    aggp1 = _sc_aggregate(h1lo.reshape(NN, DH), h1hi.reshape(NN, DH),
                          edges, zeros2, n_edges)
    h2lo, h2hi = _tc_dense(aggp1, nd2, W1, b1_row, ns2=ns2)
    aggp2 = _sc_aggregate(h2lo.reshape(NN, DH), h2hi.reshape(NN, DH),
                          edges, zeros2, n_edges)
    olo, ohi = _tc_dense(aggp2, nd2, W2, b2_row)
    out = jnp.concatenate([olo.reshape(NN, DH), ohi.reshape(NN, DH)], axis=1)
    return out[:N_NODES]


# speedup vs baseline: 12.7910x; 1.0179x over previous
import functools

import jax
import jax.numpy as jnp
from jax import lax
from jax.experimental import pallas as pl
from jax.experimental.pallas import tpu as pltpu
from jax.experimental.pallas import tpu_sc as plsc

N_NODES = 10000
D = 128
DH = D // 2

NC = 2
NS = 16
NW = NC * NS
C = 128

NN = 10240
RPT = NN // NS


_MESH = plsc.VectorSubcoreMesh(core_axis_name="c", subcore_axis_name="s")


def _sc_degrees(edges, zeros1, n_edges):
    nch_tot = n_edges // C
    nch_lo = nch_tot // NW
    n_hi = nch_tot - nch_lo * NW
    nch_hi = nch_lo + 1

    @functools.partial(
        pl.kernel,
        out_type=jax.ShapeDtypeStruct((NC, 2, NN), jnp.float32),
        mesh=_MESH,
        compiler_params=pltpu.CompilerParams(use_tc_tiling_on_sc=False),
        scratch_types=[
            pltpu.VMEM((nch_hi * C,), jnp.int32),
            pltpu.VMEM((nch_hi * C,), jnp.int32),
            pltpu.VMEM((C,), jnp.float32),
            pltpu.VMEM_SHARED((NN,), jnp.float32),
            pltpu.VMEM_SHARED((NN,), jnp.float32),
            pltpu.SemaphoreType.DMA,
            pltpu.SemaphoreType.DMA,
        ],
    )
    def deg_kernel(e_hbm, z_hbm, out_hbm,
                   src_v, dst_v, ones_v, dego_sp, degi_sp, sem_s, sem_d):
        c = lax.axis_index("c")
        s = lax.axis_index("s")
        w = c * NS + s
        pltpu.sync_copy(z_hbm.at[pl.ds(0, RPT)], dego_sp.at[pl.ds(s * RPT, RPT)])
        pltpu.sync_copy(z_hbm.at[pl.ds(0, RPT)], degi_sp.at[pl.ds(s * RPT, RPT)])
        for i in range(C // 16):
            ones_v[pl.ds(16 * i, 16)] = jnp.full((16,), 1.0, jnp.float32)

        def stage(nch, start):
            ne = nch * C
            pltpu.sync_copy(e_hbm.at[0, pl.ds(start, ne)], src_v.at[pl.ds(0, ne)])
            pltpu.sync_copy(e_hbm.at[1, pl.ds(start, ne)], dst_v.at[pl.ds(0, ne)])

        def run(nch):
            _GRP = 6

            def s_scat(g):
                return (ones_v, dego_sp.at[src_v.at[pl.ds(g * C, C)]], sem_s)

            def d_scat(g):
                return (ones_v, degi_sp.at[dst_v.at[pl.ds(g * C, C)]], sem_d)

            def body(k, carry):
                g0 = _GRP * k
                for j in range(_GRP):
                    pltpu.async_copy(*s_scat(g0 + j), add=True)
                    pltpu.async_copy(*d_scat(g0 + j), add=True)
                for j in range(_GRP):
                    pltpu.make_async_copy(*s_scat(g0 + j)).wait()
                    pltpu.make_async_copy(*d_scat(g0 + j)).wait()
                return carry

            lax.fori_loop(0, nch // _GRP, body, 0)
            for g in range(nch // _GRP * _GRP, nch):
                pltpu.sync_copy(ones_v, dego_sp.at[src_v.at[pl.ds(g * C, C)]],
                                add=True)
                pltpu.sync_copy(ones_v, degi_sp.at[dst_v.at[pl.ds(g * C, C)]],
                                add=True)

        hi = w < n_hi
        pl.when(hi)(lambda: stage(nch_hi, w * nch_hi * C))
        pl.when(~hi)(lambda: stage(nch_lo, (n_hi + w * nch_lo) * C))
        plsc.subcore_barrier()
        pl.when(hi)(lambda: run(nch_hi))
        pl.when(~hi)(lambda: run(nch_lo))
        plsc.subcore_barrier()
        pltpu.sync_copy(dego_sp.at[pl.ds(s * RPT, RPT)],
                        out_hbm.at[c, 0, pl.ds(s * RPT, RPT)])
        pltpu.sync_copy(degi_sp.at[pl.ds(s * RPT, RPT)],
                        out_hbm.at[c, 1, pl.ds(s * RPT, RPT)])

    return deg_kernel(edges, zeros1)


_NBUF = 5


def _sc_aggregate(h_lo, h_hi, edges, zeros2, n_edges):
    nch_tot = n_edges // C
    nch_lo = nch_tot // NS // _NBUF * _NBUF
    n_hi = (nch_tot - nch_lo * NS) // _NBUF
    nch_hi = nch_lo + _NBUF
    assert nch_hi * n_hi + nch_lo * (NS - n_hi) == nch_tot

    @functools.partial(
        pl.kernel,
        out_type=jax.ShapeDtypeStruct((NC, NN, DH), jnp.float32),
        mesh=_MESH,
        compiler_params=pltpu.CompilerParams(use_tc_tiling_on_sc=False),
        scratch_types=[
            pltpu.VMEM((nch_hi * C,), jnp.int32),
            pltpu.VMEM((nch_hi * C,), jnp.int32),
            pltpu.VMEM((_NBUF, C, DH), jnp.float32),
            pltpu.VMEM_SHARED((NN, DH), jnp.float32),
            [pltpu.SemaphoreType.DMA] * _NBUF,
            [pltpu.SemaphoreType.DMA] * _NBUF,
        ],
    )
    def agg_kernel(hlo_hbm, hhi_hbm, e_hbm, z_hbm, out_hbm,
                   src_v, dst_v, rows, agg_sp, sg, ss):
        c = lax.axis_index("c")
        s = lax.axis_index("s")
        pltpu.sync_copy(z_hbm, agg_sp.at[pl.ds(s * RPT, RPT)])

        def stage(nch, start):
            ne = nch * C
            pltpu.sync_copy(e_hbm.at[0, pl.ds(start, ne)], src_v.at[pl.ds(0, ne)])
            pltpu.sync_copy(e_hbm.at[1, pl.ds(start, ne)], dst_v.at[pl.ds(0, ne)])

        def run(h_hbm, nch):
            def gather(b, g):
                pltpu.async_copy(h_hbm.at[src_v.at[pl.ds(g * C, C)]],
                                 rows.at[b], sg[b])

            def gather_wait(b, g):
                pltpu.make_async_copy(h_hbm.at[src_v.at[pl.ds(g * C, C)]],
                                      rows.at[b], sg[b]).wait()

            def scatter(b, g):
                pltpu.async_copy(rows.at[b], agg_sp.at[dst_v.at[pl.ds(g * C, C)]],
                                 ss[b], add=True)

            def scatter_wait(b, g):
                pltpu.make_async_copy(rows.at[b],
                                      agg_sp.at[dst_v.at[pl.ds(g * C, C)]],
                                      ss[b]).wait()

            for b in range(_NBUF):
                gather(b, b)

            def body(k, carry):
                g0 = _NBUF * k
                for b in range(_NBUF):
                    gather_wait(b, g0 + b)
                    scatter(b, g0 + b)
                for b in range(_NBUF):
                    scatter_wait(b, g0 + b)
                    gather(b, g0 + _NBUF + b)
                return carry

            lax.fori_loop(0, nch // _NBUF - 1, body, 0)
            g0 = nch - _NBUF
            for b in range(_NBUF):
                gather_wait(b, g0 + b)
                scatter(b, g0 + b)
            for b in range(_NBUF):
                scatter_wait(b, g0 + b)

        hi = s < n_hi
        pl.when(hi)(lambda: stage(nch_hi, s * nch_hi * C))
        pl.when(~hi)(lambda: stage(nch_lo, (n_hi * _NBUF + s * nch_lo) * C))
        plsc.subcore_barrier()

        def run_core(nch):
            pl.when(c == 0)(lambda: run(hlo_hbm, nch))
            pl.when(c == 1)(lambda: run(hhi_hbm, nch))

        pl.when(hi)(lambda: run_core(nch_hi))
        pl.when(~hi)(lambda: run_core(nch_lo))

        plsc.subcore_barrier()
        pltpu.sync_copy(agg_sp.at[pl.ds(s * RPT, RPT)],
                        out_hbm.at[c, pl.ds(s * RPT, RPT)])

    return agg_kernel(h_lo, h_hi, edges, zeros2)


_BM = 1024
_BP = _BM // 2


def _tc_prep(feat_e, feat_o, deg_e, deg_o):
    def body(fe_ref, fo_ref, dge_ref, dgo_ref, lo_ref, hi_ref,
             ns_ref, nd_ref):
        ns_e = lax.rsqrt(jnp.maximum(dge_ref[0, 0] + dge_ref[1, 0], 1.0))
        ns_o = lax.rsqrt(jnp.maximum(dgo_ref[0, 0] + dgo_ref[1, 0], 1.0))
        nd_e = lax.rsqrt(jnp.maximum(dge_ref[0, 1] + dge_ref[1, 1], 1.0))
        nd_o = lax.rsqrt(jnp.maximum(dgo_ref[0, 1] + dgo_ref[1, 1], 1.0))
        ns2 = jnp.concatenate([ns_e[:, None], ns_o[:, None]], axis=1)
        nd2 = jnp.concatenate([nd_e[:, None], nd_o[:, None]], axis=1)
        he = fe_ref[...] * ns_e[:, None]
        ho = fo_ref[...] * ns_o[:, None]
        lo_ref[...] = jnp.concatenate([he[:, :DH], ho[:, :DH]], axis=1)
        hi_ref[...] = jnp.concatenate([he[:, DH:], ho[:, DH:]], axis=1)
        ns_ref[...] = ns2
        nd_ref[...] = nd2

    return pl.pallas_call(
        body,
        grid=(NN // _BM,),
        in_specs=[
            pl.BlockSpec((_BP, D), lambda i: (i, 0)),
            pl.BlockSpec((_BP, D), lambda i: (i, 0)),
            pl.BlockSpec((NC, 2, _BP), lambda i: (0, 0, i)),
            pl.BlockSpec((NC, 2, _BP), lambda i: (0, 0, i)),
        ],
        out_specs=[
            pl.BlockSpec((_BP, D), lambda i: (i, 0)),
            pl.BlockSpec((_BP, D), lambda i: (i, 0)),
            pl.BlockSpec((_BP, 2), lambda i: (i, 0)),
            pl.BlockSpec((_BP, 2), lambda i: (i, 0)),
        ],
        out_shape=[
            jax.ShapeDtypeStruct((NN // 2, D), jnp.float32),
            jax.ShapeDtypeStruct((NN // 2, D), jnp.float32),
            jax.ShapeDtypeStruct((NN // 2, 2), jnp.float32),
            jax.ShapeDtypeStruct((NN // 2, 2), jnp.float32),
        ],
    )(feat_e, feat_o, deg_e, deg_o)


def _tc_dense(aggp, nd2, W, b_row, ns2=None):
    def body(a_ref, nd_ref, w_ref, b_ref, *rest):
        if ns2 is None:
            lo_ref, hi_ref = rest
        else:
            ns_ref, lo_ref, hi_ref = rest
        nde, ndo = nd_ref[:, 0:1], nd_ref[:, 1:2]
        a0, a1 = a_ref[0], a_ref[1]
        b = b_ref[...]
        ye = (jnp.dot(a0[:, :DH] * nde, w_ref[:DH, :],
                      preferred_element_type=jnp.float32)
              + jnp.dot(a1[:, :DH] * nde, w_ref[DH:, :],
                        preferred_element_type=jnp.float32) + b)
        yo = (jnp.dot(a0[:, DH:] * ndo, w_ref[:DH, :],
                      preferred_element_type=jnp.float32)
              + jnp.dot(a1[:, DH:] * ndo, w_ref[DH:, :],
                        preferred_element_type=jnp.float32) + b)
        if ns2 is not None:
            ye = jnp.maximum(ye, 0.0) * ns_ref[:, 0:1]
            yo = jnp.maximum(yo, 0.0) * ns_ref[:, 1:2]
        lo_ref[...] = jnp.concatenate([ye[:, :DH], yo[:, :DH]], axis=1)
        hi_ref[...] = jnp.concatenate([ye[:, DH:], yo[:, DH:]], axis=1)

    in_specs = [
        pl.BlockSpec((NC, _BP, D), lambda i: (0, i, 0)),
        pl.BlockSpec((_BP, 2), lambda i: (i, 0)),
        pl.BlockSpec((D, D), lambda i: (0, 0)),
        pl.BlockSpec((1, D), lambda i: (0, 0)),
    ]
    args = [aggp.reshape(NC, NN // 2, D), nd2, W, b_row]
    if ns2 is not None:
        in_specs.append(pl.BlockSpec((_BP, 2), lambda i: (i, 0)))
        args.append(ns2)
    return pl.pallas_call(
        body,
        grid=(NN // _BM,),
        in_specs=in_specs,
        out_specs=[pl.BlockSpec((_BP, D), lambda i: (i, 0))] * 2,
        out_shape=[jax.ShapeDtypeStruct((NN // 2, D), jnp.float32)] * 2,
    )(*args)


def kernel(feat_data, edge_index, W1, b1, W2, b2):
    n_edges = edge_index.shape[1]
    edges = edge_index.astype(jnp.int32)

    zeros1 = jnp.zeros((NN,), jnp.float32)
    zeros2 = jnp.zeros((RPT, DH), jnp.float32)
    b1_row = b1.reshape(1, D)
    b2_row = b2.reshape(1, D)

    degp = _sc_degrees(edges, zeros1, n_edges)

    feat_pad = jnp.pad(feat_data, ((0, NN - N_NODES), (0, 0)))
    h1lo, h1hi, ns2, nd2 = _tc_prep(feat_pad[0::2], feat_pad[1::2],
                                    degp[:, :, 0::2], degp[:, :, 1::2])
    aggp1 = _sc_aggregate(h1lo.reshape(NN, DH), h1hi.reshape(NN, DH),
                          edges, zeros2, n_edges)
    h2lo, h2hi = _tc_dense(aggp1, nd2, W1, b1_row, ns2=ns2)
    aggp2 = _sc_aggregate(h2lo.reshape(NN, DH), h2hi.reshape(NN, DH),
                          edges, zeros2, n_edges)
    olo, ohi = _tc_dense(aggp2, nd2, W2, b2_row)
    out = jnp.concatenate([olo.reshape(NN, DH), ohi.reshape(NN, DH)], axis=1)
    return out[:N_NODES]
